# Initial kernel scaffold; baseline (speedup 1.0000x reference)
import jax
import jax.numpy as jnp
from jax.experimental import pallas as pl


def kernel(x, pos, edge_index, ln_w1, ln_b1, ln_w2, ln_b2, gn_w1, gn_b1, gn_w2, gn_b2, gn_w3, gn_b3, gat0_w, gat0_asrc, gat0_adst, gat0_b, gat1_w, gat1_asrc, gat1_adst, gat1_b, gat2_w, gat2_asrc, gat2_adst, gat2_b, gat3_w, gat3_asrc, gat3_adst, gat3_b, gat4_w, gat4_asrc, gat4_adst, gat4_b, gcn_w, gcn_b):
    raise NotImplementedError("write your pallas kernel here")


# speedup vs baseline: 1.1314x; 1.1314x over previous
import functools

import jax
import jax.numpy as jnp
from jax.experimental import pallas as pl


def _mm_body(x_ref, w_ref, b_ref, o_ref, *, act_in, act_out):
    x = x_ref[...]
    if act_in == "relu":
        x = jnp.maximum(x, 0.0)
    y = jnp.dot(x, w_ref[...], preferred_element_type=jnp.float32)
    y = y + b_ref[...]
    if act_out == "relu":
        y = jnp.maximum(y, 0.0)
    o_ref[...] = y


def _mm(x, w, b, act_in="none", act_out="none", bm=1000):
    m, k = x.shape
    n = w.shape[1]
    assert m % bm == 0, (m, bm)
    if b is None:
        b = jnp.zeros((n,), jnp.float32)
    b2 = b.reshape(1, n)
    return pl.pallas_call(
        functools.partial(_mm_body, act_in=act_in, act_out=act_out),
        grid=(m // bm,),
        in_specs=[
            pl.BlockSpec((bm, k), lambda i: (i, 0)),
            pl.BlockSpec((k, n), lambda i: (0, 0)),
            pl.BlockSpec((1, n), lambda i: (0, 0)),
        ],
        out_specs=pl.BlockSpec((bm, n), lambda i: (i, 0)),
        out_shape=jax.ShapeDtypeStruct((m, n), jnp.float32),
    )(x, w, b2)


def kernel(x, pos, edge_index, ln_w1, ln_b1, ln_w2, ln_b2, gn_w1, gn_b1, gn_w2, gn_b2, gn_w3, gn_b3, gat0_w, gat0_asrc, gat0_adst, gat0_b, gat1_w, gat1_asrc, gat1_adst, gat1_b, gat2_w, gat2_asrc, gat2_adst, gat2_b, gat3_w, gat3_asrc, gat3_adst, gat3_b, gat4_w, gat4_asrc, gat4_adst, gat4_b, gcn_w, gcn_b):
    n = x.shape[0]
    e = edge_index.shape[1]
    loop = jnp.arange(n, dtype=edge_index.dtype)
    src = jnp.concatenate([edge_index[0], loop])
    dst = jnp.concatenate([edge_index[1], loop])
    et = e + n
    et_pad = ((et + 999) // 1000) * 1000

    gats = [
        (gat0_w, gat0_asrc, gat0_adst, gat0_b),
        (gat1_w, gat1_asrc, gat1_adst, gat1_b),
        (gat2_w, gat2_asrc, gat2_adst, gat2_b),
        (gat3_w, gat3_asrc, gat3_adst, gat3_b),
        (gat4_w, gat4_asrc, gat4_adst, gat4_b),
    ]

    in_x = x.shape[1]
    w1x = ln_w1[:in_x]
    w1p = ln_w1[in_x:]
    xw = _mm(x, w1x, ln_b1)
    pw = _mm(pos, w1p, None)
    a_node = xw + pw
    pre = a_node[src] - pw[dst]
    pre = jnp.pad(pre, ((0, et_pad - et), (0, 0)))
    msg = _mm(pre, ln_w2, ln_b2, act_in="relu")[:et]
    agg = jax.ops.segment_max(msg, dst, num_segments=n)
    agg = jnp.where(jnp.isfinite(agg), agg, 0.0)

    h = _mm(agg, gn_w1, gn_b1, act_out="relu")
    h = _mm(h, gn_w2, gn_b2, act_out="relu")
    h = _mm(h, gn_w3, gn_b3)

    for (w, asrc, adst, b) in gats:
        do = w.shape[1]
        xp = _mm(h, w, None)
        att = _mm(xp, jnp.stack([asrc, adst], axis=1), None)
        als, ald = att[:, 0], att[:, 1]
        ee = als[src] + ald[dst]
        ee = jnp.where(ee > 0, ee, 0.2 * ee)
        m = jax.ops.segment_max(ee, dst, num_segments=n)
        m = jnp.where(jnp.isfinite(m), m, 0.0)
        ex = jnp.exp(ee - m[dst])
        s = jax.ops.segment_sum(ex, dst, num_segments=n)
        alpha = ex / (s[dst] + 1e-16)
        h = jax.ops.segment_sum(xp[src] * alpha[:, None], dst, num_segments=n) + b
        h = jnp.maximum(h, 0.0)

    xp = _mm(h, gcn_w, None)
    deg = jax.ops.segment_sum(jnp.ones_like(dst, dtype=jnp.float32), dst,
                              num_segments=n)
    dinv = jnp.where(deg > 0, 1.0 / jnp.sqrt(deg), 0.0)
    norm = dinv[src] * dinv[dst]
    out = jax.ops.segment_sum(xp[src] * norm[:, None], dst, num_segments=n) + gcn_b
    return out
